# baseline (speedup 1.0000x reference)
import jax
import jax.numpy as jnp
from jax import lax
from jax.experimental import pallas as pl
from jax.experimental.pallas import tpu as pltpu
from jax.experimental.pallas import tpu_sc as plsc

N = 10000
F1 = 128
F2 = 64
E = 320000

NC = 2
NS = 16
NW = NC * NS
EW = E // NW
K = 80
NCHUNK = EW // K
NP = 10240
ROWS_PER_SUB = NP // NS


NB = 5
NBLK = NCHUNK // NB
FW = 64


def _make_spmm(P):
    mesh = plsc.VectorSubcoreMesh(core_axis_name="c", subcore_axis_name="s")

    def body(src_hbm, dst_hbm, w_hbm, tab_hbm, out_hbm,
             src_all, dst_all, w_all,
             ix0, ix1, ix2, ix3, ix4,
             rows0, rows1, rows2, rows3, rows4,
             sc0, sc1, sc2, sc3, sc4, acc,
             g0, g1, g2, g3, g4, s0, s1, s2, s3, s4):
        rows = [rows0, rows1, rows2, rows3, rows4]
        scl = [sc0, sc1, sc2, sc3, sc4]
        ixs = [ix0, ix1, ix2, ix3, ix4]
        gsem = [g0, g1, g2, g3, g4]
        ssem = [s0, s1, s2, s3, s4]
        cid = lax.axis_index("c")
        sid = lax.axis_index("s")
        wid = sid * NC + cid

        pltpu.sync_copy(src_hbm.at[wid], src_all)
        pltpu.sync_copy(dst_hbm.at[wid], dst_all)
        pltpu.sync_copy(w_hbm.at[wid], w_all)

        def fire_gather(c, b, p):
            if P == 1:
                idx = src_all.at[pl.ds(c * K, K)]
            else:
                @plsc.parallel_loop(0, K // 16)
                def _mkidx(g):
                    sv = src_all[pl.ds(c * K + g * 16, 16)]
                    ixs[b][pl.ds(g * 16, 16)] = sv * P + p
                idx = ixs[b]
            pltpu.async_copy(tab_hbm.at[idx], rows[b], gsem[b])

        def wait_gather(c, b):
            pltpu.make_async_copy(tab_hbm.at[src_all.at[pl.ds(c * K, K)]],
                                  rows[b], gsem[b]).wait()

        def scale(c, b):
            @plsc.parallel_loop(0, K // 16, unroll=4)
            def group(g):
                wvec = w_all[pl.ds(c * K + g * 16, 16)]
                for t in range(16):
                    e = g * 16 + t
                    wv = wvec[t]
                    for j in range(FW // 16):
                        sl = pl.ds(j * 16, 16)
                        scl[b][e, sl] = rows[b][e, sl] * wv

        for p in range(P):
            def zrow(r, c):
                for j in range(FW // 16):
                    rows0[r, pl.ds(j * 16, 16)] = jnp.zeros((16,), jnp.float32)
                return c
            lax.fori_loop(0, K, zrow, 0)
            for t in range(ROWS_PER_SUB // K):
                pltpu.sync_copy(
                    rows0, acc.at[pl.ds(sid * ROWS_PER_SUB + t * K, K)])
            plsc.subcore_barrier()

            for b in range(NB):
                fire_gather(b, b, p)

            def blk(i, carry):
                for b in range(NB):
                    c = i * NB + b
                    wait_gather(c, b)
                    scale(c, b)
                    pltpu.async_copy(scl[b], acc.at[dst_all.at[c]], ssem[b],
                                     add=True)

                @pl.when(i < NBLK - 1)
                def _():
                    for b in range(NB):
                        cn = (i + 1) * NB + b
                        pltpu.make_async_copy(scl[b], acc.at[dst_all.at[cn]],
                                              ssem[b]).wait()
                        fire_gather(cn, b, p)
                return carry
            lax.fori_loop(0, NBLK, blk, 0)

            for b in range(NB):
                pltpu.make_async_copy(
                    scl[b], acc.at[dst_all.at[NCHUNK - NB + b]],
                    ssem[b]).wait()
            plsc.subcore_barrier()
            pltpu.sync_copy(
                acc.at[pl.ds(sid * ROWS_PER_SUB, ROWS_PER_SUB)],
                out_hbm.at[p, cid, pl.ds(sid * ROWS_PER_SUB, ROWS_PER_SUB)])
            if p + 1 < P:
                plsc.subcore_barrier()

    return pl.kernel(
        body,
        out_type=jax.ShapeDtypeStruct((P, NC, NP, FW), jnp.float32),
        mesh=mesh,
        scratch_types=[
            pltpu.VMEM((EW,), jnp.int32),
            pltpu.VMEM((NCHUNK, K), jnp.int32),
            pltpu.VMEM((EW,), jnp.float32),
        ] + [pltpu.VMEM((K,), jnp.int32)] * NB + [
            pltpu.VMEM((K, FW), jnp.float32)] * (2 * NB) + [
            pltpu.VMEM_SHARED((NP, FW), jnp.float32),
        ] + [pltpu.SemaphoreType.DMA] * (2 * NB),
        compiler_params=pltpu.CompilerParams(use_tc_tiling_on_sc=False),
    )


_spmm1 = _make_spmm(2)
_spmm2 = _make_spmm(1)


def _mm_body(x_ref, w_ref, o_ref):
    o_ref[...] = jnp.dot(x_ref[...], w_ref[...],
                         preferred_element_type=jnp.float32)


def _tc_mm(x, w):
    return pl.pallas_call(
        _mm_body,
        out_shape=jax.ShapeDtypeStruct((x.shape[0], w.shape[1]), jnp.float32),
    )(x, w)


def _mid_body(p_ref, b1_ref, w2_ref, o_ref):
    pv = p_ref[...]
    h0 = pv[0, 0, :N] + pv[0, 1, :N]
    h1 = pv[1, 0, :N] + pv[1, 1, :N]
    h = jnp.concatenate([h0, h1], axis=1) + b1_ref[...]
    h = jnp.maximum(h, 0.0)
    o_ref[...] = jnp.dot(h, w2_ref[...], preferred_element_type=jnp.float32)


def _tc_mid(p, b1, w2):
    return pl.pallas_call(
        _mid_body,
        out_shape=jax.ShapeDtypeStruct((N, F2), jnp.float32),
    )(p, b1, w2)


def _out_body(p_ref, b2_ref, o_ref):
    pv = p_ref[...]
    z = pv[0, :N] + pv[1, :N] + b2_ref[...]
    m = jnp.max(z, axis=1, keepdims=True)
    zs = z - m
    o_ref[...] = zs - jnp.log(jnp.sum(jnp.exp(zs), axis=1, keepdims=True))


def _tc_out(p, b2):
    return pl.pallas_call(
        _out_body,
        out_shape=jax.ShapeDtypeStruct((N, F2), jnp.float32),
    )(p, b2)


@jax.jit
def kernel(x, edge_index, edge_weight, W1, b1, W2, b2):
    src = edge_index[1].reshape(NW, EW)
    dst = edge_index[0].reshape(NW, NCHUNK, K)
    w = edge_weight.reshape(NW, EW)

    support = _tc_mm(x, W1)
    tab1 = support.reshape(2 * N, FW)
    p1 = _spmm1(src, dst, w, tab1)
    s2 = _tc_mid(p1, b1.reshape(1, F1), W2)
    p2 = _spmm2(src, dst, w, s2)
    return _tc_out(p2[0], b2.reshape(1, F2))


# speedup vs baseline: 1.3643x; 1.3643x over previous
import jax
import jax.numpy as jnp
from jax import lax
from jax.experimental import pallas as pl
from jax.experimental.pallas import tpu as pltpu
from jax.experimental.pallas import tpu_sc as plsc

N = 10000
F1 = 128
F2 = 64
E = 320000

NC = 2
NS = 16
NW = NC * NS
EW = E // NW
K = 80
NCHUNK = EW // K
NP = 10240
ROWS_PER_SUB = NP // NS


NB = 5
NBLK = NCHUNK // NB
FW = 64


def _make_spmm(P):
    mesh = plsc.VectorSubcoreMesh(core_axis_name="c", subcore_axis_name="s")

    def body(src_hbm, dst_hbm, w_hbm, tab_hbm, out_hbm,
             src_all, dst_all, w_all,
             ix0, ix1, ix2, ix3, ix4,
             rows0, rows1, rows2, rows3, rows4,
             sc0, sc1, sc2, sc3, sc4, acc,
             g0, g1, g2, g3, g4, s0, s1, s2, s3, s4):
        rows = [rows0, rows1, rows2, rows3, rows4]
        scl = [sc0, sc1, sc2, sc3, sc4]
        ixs = [ix0, ix1, ix2, ix3, ix4]
        gsem = [g0, g1, g2, g3, g4]
        ssem = [s0, s1, s2, s3, s4]
        cid = lax.axis_index("c")
        sid = lax.axis_index("s")
        wid = sid * NC + cid

        pltpu.sync_copy(src_hbm.at[wid], src_all)
        pltpu.sync_copy(dst_hbm.at[wid], dst_all)
        pltpu.sync_copy(w_hbm.at[wid], w_all)

        def fire_gather(c, b, p):
            if P == 1:
                idx = src_all.at[pl.ds(c * K, K)]
            else:
                @plsc.parallel_loop(0, K // 16)
                def _mkidx(g):
                    sv = src_all[pl.ds(c * K + g * 16, 16)]
                    ixs[b][pl.ds(g * 16, 16)] = sv * P + p
                idx = ixs[b]
            pltpu.async_copy(tab_hbm.at[idx], rows[b], gsem[b])

        def wait_gather(c, b):
            pltpu.make_async_copy(tab_hbm.at[src_all.at[pl.ds(c * K, K)]],
                                  rows[b], gsem[b]).wait()

        def scale(c, b):
            @plsc.parallel_loop(0, K // 16, unroll=2)
            def group(g):
                wvec = w_all[pl.ds(c * K + g * 16, 16)]
                for t in range(16):
                    e = g * 16 + t
                    wv = wvec[t]
                    for j in range(FW // 16):
                        sl = pl.ds(j * 16, 16)
                        scl[b][e, sl] = rows[b][e, sl] * wv

        for p in range(P):
            def zrow(r, c):
                for j in range(FW // 16):
                    rows0[r, pl.ds(j * 16, 16)] = jnp.zeros((16,), jnp.float32)
                return c
            lax.fori_loop(0, K, zrow, 0)
            for t in range(ROWS_PER_SUB // K):
                pltpu.sync_copy(
                    rows0, acc.at[pl.ds(sid * ROWS_PER_SUB + t * K, K)])
            plsc.subcore_barrier()

            for b in range(NB):
                fire_gather(b, b, p)

            def blk(i, carry):
                for b in range(NB):
                    c = i * NB + b
                    wait_gather(c, b)
                    scale(c, b)
                    pltpu.async_copy(scl[b], acc.at[dst_all.at[c]], ssem[b],
                                     add=True)

                @pl.when(i < NBLK - 1)
                def _():
                    for b in range(NB):
                        cn = (i + 1) * NB + b
                        pltpu.make_async_copy(scl[b], acc.at[dst_all.at[cn]],
                                              ssem[b]).wait()
                        fire_gather(cn, b, p)
                return carry
            lax.fori_loop(0, NBLK, blk, 0)

            for b in range(NB):
                pltpu.make_async_copy(
                    scl[b], acc.at[dst_all.at[NCHUNK - NB + b]],
                    ssem[b]).wait()
            plsc.subcore_barrier()
            pltpu.sync_copy(
                acc.at[pl.ds(sid * ROWS_PER_SUB, ROWS_PER_SUB)],
                out_hbm.at[p, cid, pl.ds(sid * ROWS_PER_SUB, ROWS_PER_SUB)])
            if p + 1 < P:
                plsc.subcore_barrier()

    return pl.kernel(
        body,
        out_type=jax.ShapeDtypeStruct((P, NC, NP, FW), jnp.float32),
        mesh=mesh,
        scratch_types=[
            pltpu.VMEM((EW,), jnp.int32),
            pltpu.VMEM((NCHUNK, K), jnp.int32),
            pltpu.VMEM((EW,), jnp.float32),
        ] + [pltpu.VMEM((K,), jnp.int32)] * NB + [
            pltpu.VMEM((K, FW), jnp.float32)] * (2 * NB) + [
            pltpu.VMEM_SHARED((NP, FW), jnp.float32),
        ] + [pltpu.SemaphoreType.DMA] * (2 * NB),
        compiler_params=pltpu.CompilerParams(use_tc_tiling_on_sc=False),
    )


_spmm1 = _make_spmm(2)
_spmm2 = _make_spmm(1)


def _mm_body(x_ref, w_ref, o_ref):
    o_ref[...] = jnp.dot(x_ref[...], w_ref[...],
                         preferred_element_type=jnp.float32)


def _tc_mm(x, w):
    return pl.pallas_call(
        _mm_body,
        out_shape=jax.ShapeDtypeStruct((x.shape[0], w.shape[1]), jnp.float32),
    )(x, w)


def _mid_body(p_ref, b1_ref, w2_ref, o_ref):
    pv = p_ref[...]
    h0 = pv[0, 0, :N] + pv[0, 1, :N]
    h1 = pv[1, 0, :N] + pv[1, 1, :N]
    h = jnp.concatenate([h0, h1], axis=1) + b1_ref[...]
    h = jnp.maximum(h, 0.0)
    o_ref[...] = jnp.dot(h, w2_ref[...], preferred_element_type=jnp.float32)


def _tc_mid(p, b1, w2):
    return pl.pallas_call(
        _mid_body,
        out_shape=jax.ShapeDtypeStruct((N, F2), jnp.float32),
    )(p, b1, w2)


def _out_body(p_ref, b2_ref, o_ref):
    pv = p_ref[...]
    z = pv[0, :N] + pv[1, :N] + b2_ref[...]
    m = jnp.max(z, axis=1, keepdims=True)
    zs = z - m
    o_ref[...] = zs - jnp.log(jnp.sum(jnp.exp(zs), axis=1, keepdims=True))


def _tc_out(p, b2):
    return pl.pallas_call(
        _out_body,
        out_shape=jax.ShapeDtypeStruct((N, F2), jnp.float32),
    )(p, b2)


@jax.jit
def kernel(x, edge_index, edge_weight, W1, b1, W2, b2):
    src = edge_index[1].reshape(NW, EW)
    dst = edge_index[0].reshape(NW, NCHUNK, K)
    w = edge_weight.reshape(NW, EW)

    support = _tc_mm(x, W1)
    tab1 = support.reshape(2 * N, FW)
    p1 = _spmm1(src, dst, w, tab1)
    s2 = _tc_mid(p1, b1.reshape(1, F1), W2)
    p2 = _spmm2(src, dst, w, s2)
    return _tc_out(p2[0], b2.reshape(1, F2))


# speedup vs baseline: 1.3740x; 1.0071x over previous
import jax
import jax.numpy as jnp
from jax import lax
from jax.experimental import pallas as pl
from jax.experimental.pallas import tpu as pltpu
from jax.experimental.pallas import tpu_sc as plsc

N = 10000
F1 = 128
F2 = 64
E = 320000

NC = 2
NS = 16
NW = NC * NS
EW = E // NW
K = 80
NCHUNK = EW // K
NP = 10240
ROWS_PER_SUB = NP // NS


NB = 5
NBLK = NCHUNK // NB
FW = 64


def _make_spmm(P):
    mesh = plsc.VectorSubcoreMesh(core_axis_name="c", subcore_axis_name="s")

    def body(ei_hbm, w_hbm, tab_hbm, out_hbm,
             src_all, dst_all, w_all,
             ix0, ix1, ix2, ix3, ix4,
             dx0, dx1, dx2, dx3, dx4,
             rows0, rows1, rows2, rows3, rows4,
             sc0, sc1, sc2, sc3, sc4, acc,
             g0, g1, g2, g3, g4, s0, s1, s2, s3, s4):
        rows = [rows0, rows1, rows2, rows3, rows4]
        scl = [sc0, sc1, sc2, sc3, sc4]
        ixs = [ix0, ix1, ix2, ix3, ix4]
        dxs = [dx0, dx1, dx2, dx3, dx4]
        gsem = [g0, g1, g2, g3, g4]
        ssem = [s0, s1, s2, s3, s4]
        cid = lax.axis_index("c")
        sid = lax.axis_index("s")
        wid = sid * NC + cid

        ebase = wid * EW
        pltpu.sync_copy(ei_hbm.at[1, pl.ds(ebase, EW)], src_all)
        pltpu.sync_copy(ei_hbm.at[0, pl.ds(ebase, EW)], dst_all)
        pltpu.sync_copy(w_hbm.at[pl.ds(ebase, EW)], w_all)

        def fire_gather(c, b, p):
            if P == 1:
                idx = src_all.at[pl.ds(c * K, K)]
            else:
                @plsc.parallel_loop(0, K // 16)
                def _mkidx(g):
                    sv = src_all[pl.ds(c * K + g * 16, 16)]
                    ixs[b][pl.ds(g * 16, 16)] = sv * P + p
                idx = ixs[b]
            pltpu.async_copy(tab_hbm.at[idx], rows[b], gsem[b])

        def wait_gather(c, b):
            pltpu.make_async_copy(tab_hbm.at[src_all.at[pl.ds(c * K, K)]],
                                  rows[b], gsem[b]).wait()

        def scale(c, b):
            @plsc.parallel_loop(0, K // 16, unroll=2)
            def group(g):
                wvec = w_all[pl.ds(c * K + g * 16, 16)]
                for t in range(16):
                    e = g * 16 + t
                    wv = wvec[t]
                    for j in range(FW // 16):
                        sl = pl.ds(j * 16, 16)
                        scl[b][e, sl] = rows[b][e, sl] * wv

        for p in range(P):
            def zrow(r, c):
                for j in range(FW // 16):
                    rows0[r, pl.ds(j * 16, 16)] = jnp.zeros((16,), jnp.float32)
                return c
            lax.fori_loop(0, K, zrow, 0)
            for t in range(ROWS_PER_SUB // K):
                pltpu.sync_copy(
                    rows0, acc.at[pl.ds(sid * ROWS_PER_SUB + t * K, K)])
            plsc.subcore_barrier()

            for b in range(NB):
                fire_gather(b, b, p)

            def blk(i, carry):
                for b in range(NB):
                    c = i * NB + b
                    wait_gather(c, b)
                    scale(c, b)

                    @plsc.parallel_loop(0, K // 16)
                    def _mkdst(g):
                        dxs[b][pl.ds(g * 16, 16)] = \
                            dst_all[pl.ds(c * K + g * 16, 16)]
                    pltpu.async_copy(scl[b], acc.at[dxs[b]], ssem[b],
                                     add=True)

                @pl.when(i < NBLK - 1)
                def _():
                    for b in range(NB):
                        cn = (i + 1) * NB + b
                        pltpu.make_async_copy(scl[b], acc.at[dxs[b]],
                                              ssem[b]).wait()
                        fire_gather(cn, b, p)
                return carry
            lax.fori_loop(0, NBLK, blk, 0)

            for b in range(NB):
                pltpu.make_async_copy(scl[b], acc.at[dxs[b]], ssem[b]).wait()
            plsc.subcore_barrier()
            pltpu.sync_copy(
                acc.at[pl.ds(sid * ROWS_PER_SUB, ROWS_PER_SUB)],
                out_hbm.at[p, cid, pl.ds(sid * ROWS_PER_SUB, ROWS_PER_SUB)])
            if p + 1 < P:
                plsc.subcore_barrier()

    return pl.kernel(
        body,
        out_type=jax.ShapeDtypeStruct((P, NC, NP, FW), jnp.float32),
        mesh=mesh,
        scratch_types=[
            pltpu.VMEM((EW,), jnp.int32),
            pltpu.VMEM((EW,), jnp.int32),
            pltpu.VMEM((EW,), jnp.float32),
        ] + [pltpu.VMEM((K,), jnp.int32)] * (2 * NB) + [
            pltpu.VMEM((K, FW), jnp.float32)] * (2 * NB) + [
            pltpu.VMEM_SHARED((NP, FW), jnp.float32),
        ] + [pltpu.SemaphoreType.DMA] * (2 * NB),
        compiler_params=pltpu.CompilerParams(use_tc_tiling_on_sc=False),
    )


_spmm1 = _make_spmm(2)
_spmm2 = _make_spmm(1)


def _mm_body(x_ref, w_ref, o_ref):
    o_ref[...] = jnp.dot(x_ref[...], w_ref[...],
                         preferred_element_type=jnp.float32)


def _tc_mm(x, w):
    return pl.pallas_call(
        _mm_body,
        out_shape=jax.ShapeDtypeStruct((x.shape[0], w.shape[1]), jnp.float32),
    )(x, w)


def _mid_body(p_ref, b1_ref, w2_ref, o_ref):
    pv = p_ref[...]
    h0 = pv[0, 0, :N] + pv[0, 1, :N]
    h1 = pv[1, 0, :N] + pv[1, 1, :N]
    h = jnp.concatenate([h0, h1], axis=1) + b1_ref[...]
    h = jnp.maximum(h, 0.0)
    o_ref[...] = jnp.dot(h, w2_ref[...], preferred_element_type=jnp.float32)


def _tc_mid(p, b1, w2):
    return pl.pallas_call(
        _mid_body,
        out_shape=jax.ShapeDtypeStruct((N, F2), jnp.float32),
    )(p, b1, w2)


def _out_body(p_ref, b2_ref, o_ref):
    pv = p_ref[...]
    z = pv[0, :N] + pv[1, :N] + b2_ref[...]
    m = jnp.max(z, axis=1, keepdims=True)
    zs = z - m
    o_ref[...] = zs - jnp.log(jnp.sum(jnp.exp(zs), axis=1, keepdims=True))


def _tc_out(p, b2):
    return pl.pallas_call(
        _out_body,
        out_shape=jax.ShapeDtypeStruct((N, F2), jnp.float32),
    )(p, b2)


@jax.jit
def kernel(x, edge_index, edge_weight, W1, b1, W2, b2):
    support = _tc_mm(x, W1)
    tab1 = support.reshape(2 * N, FW)
    p1 = _spmm1(edge_index, edge_weight, tab1)
    s2 = _tc_mid(p1, b1.reshape(1, F1), W2)
    p2 = _spmm2(edge_index, edge_weight, s2)
    return _tc_out(p2[0], b2.reshape(1, F2))


# speedup vs baseline: 1.5322x; 1.1151x over previous
import jax
import jax.numpy as jnp
from jax import lax
from jax.experimental import pallas as pl
from jax.experimental.pallas import tpu as pltpu
from jax.experimental.pallas import tpu_sc as plsc

N = 10000
F1 = 128
F2 = 64
E = 320000

NC = 2
NS = 16
NW = NC * NS
EW = E // NW
K = 80
NCHUNK = EW // K
NP = 10240
ROWS_PER_SUB = NP // NS


NB = 5
NBLK = NCHUNK // NB
FW = 64


def _make_spmm(P):
    mesh = plsc.VectorSubcoreMesh(core_axis_name="c", subcore_axis_name="s")

    def body(ei_hbm, w_hbm, tab_hbm, out_hbm,
             src_all, dst_all, w_all,
             ix0, ix1, ix2, ix3, ix4,
             dx0, dx1, dx2, dx3, dx4,
             rows0, rows1, rows2, rows3, rows4,
             sc0, sc1, sc2, sc3, sc4, acc,
             g0, g1, g2, g3, g4, s0, s1, s2, s3, s4):
        rows = [rows0, rows1, rows2, rows3, rows4]
        scl = [sc0, sc1, sc2, sc3, sc4]
        ixs = [ix0, ix1, ix2, ix3, ix4]
        dxs = [dx0, dx1, dx2, dx3, dx4]
        gsem = [g0, g1, g2, g3, g4]
        ssem = [s0, s1, s2, s3, s4]
        cid = lax.axis_index("c")
        sid = lax.axis_index("s")
        wid = sid * NC + cid

        ebase = wid * EW
        pltpu.sync_copy(ei_hbm.at[1, pl.ds(ebase, EW)], src_all)
        pltpu.sync_copy(ei_hbm.at[0, pl.ds(ebase, EW)], dst_all)
        pltpu.sync_copy(w_hbm.at[pl.ds(ebase, EW)], w_all)

        def fire_gather(c, b, p):
            if P == 1:
                idx = src_all.at[pl.ds(c * K, K)]
            else:
                @plsc.parallel_loop(0, K // 16)
                def _mkidx(g):
                    sv = src_all[pl.ds(c * K + g * 16, 16)]
                    ixs[b][pl.ds(g * 16, 16)] = sv * P + p
                idx = ixs[b]
            pltpu.async_copy(tab_hbm.at[idx], rows[b], gsem[b])

        def wait_gather(c, b):
            pltpu.make_async_copy(tab_hbm.at[src_all.at[pl.ds(c * K, K)]],
                                  rows[b], gsem[b]).wait()

        def scale(c, b):
            @plsc.parallel_loop(0, K // 16, unroll=2)
            def group(g):
                wvec = w_all[pl.ds(c * K + g * 16, 16)]
                for t in range(16):
                    e = g * 16 + t
                    wv = wvec[t]
                    for j in range(FW // 16):
                        sl = pl.ds(j * 16, 16)
                        scl[b][e, sl] = rows[b][e, sl] * wv

        for p in range(P):
            def zrow(r, c):
                for j in range(FW // 16):
                    rows0[r, pl.ds(j * 16, 16)] = jnp.zeros((16,), jnp.float32)
                return c
            lax.fori_loop(0, K, zrow, 0)
            for t in range(ROWS_PER_SUB // K):
                pltpu.sync_copy(
                    rows0, acc.at[pl.ds(sid * ROWS_PER_SUB + t * K, K)])
            plsc.subcore_barrier()

            for b in range(NB):
                fire_gather(b, b, p)

            def blk(i, carry):
                for b in range(NB):
                    c = i * NB + b
                    wait_gather(c, b)
                    scale(c, b)

                    @plsc.parallel_loop(0, K // 16)
                    def _mkdst(g):
                        dxs[b][pl.ds(g * 16, 16)] = \
                            dst_all[pl.ds(c * K + g * 16, 16)]
                    pltpu.async_copy(scl[b], acc.at[dxs[b]], ssem[b],
                                     add=True)

                @pl.when(i < NBLK - 1)
                def _():
                    for b in range(NB):
                        cn = (i + 1) * NB + b
                        pltpu.make_async_copy(scl[b], acc.at[dxs[b]],
                                              ssem[b]).wait()
                        fire_gather(cn, b, p)
                return carry
            lax.fori_loop(0, NBLK, blk, 0)

            for b in range(NB):
                pltpu.make_async_copy(scl[b], acc.at[dxs[b]], ssem[b]).wait()
            plsc.subcore_barrier()
            dst_slice = (out_hbm.at[cid, pl.ds(sid * ROWS_PER_SUB, ROWS_PER_SUB)]
                         if P == 1 else
                         out_hbm.at[p, cid, pl.ds(sid * ROWS_PER_SUB, ROWS_PER_SUB)])
            pltpu.sync_copy(
                acc.at[pl.ds(sid * ROWS_PER_SUB, ROWS_PER_SUB)], dst_slice)
            if p + 1 < P:
                plsc.subcore_barrier()

    return pl.kernel(
        body,
        out_type=jax.ShapeDtypeStruct(
            (NC, NP, FW) if P == 1 else (P, NC, NP, FW), jnp.float32),
        mesh=mesh,
        scratch_types=[
            pltpu.VMEM((EW,), jnp.int32),
            pltpu.VMEM((EW,), jnp.int32),
            pltpu.VMEM((EW,), jnp.float32),
        ] + [pltpu.VMEM((K,), jnp.int32)] * (2 * NB) + [
            pltpu.VMEM((K, FW), jnp.float32)] * (2 * NB) + [
            pltpu.VMEM_SHARED((NP, FW), jnp.float32),
        ] + [pltpu.SemaphoreType.DMA] * (2 * NB),
        compiler_params=pltpu.CompilerParams(use_tc_tiling_on_sc=False),
    )


_spmm1 = _make_spmm(2)
_spmm2 = _make_spmm(1)


def _mm_body(x_ref, w_ref, o_ref):
    o_ref[...] = jnp.dot(x_ref[...], w_ref[...],
                         preferred_element_type=jnp.float32)


def _tc_mm(x, w):
    return pl.pallas_call(
        _mm_body,
        out_shape=jax.ShapeDtypeStruct((x.shape[0], w.shape[1]), jnp.float32),
    )(x, w)


def _mid_body(p_ref, ba_ref, bb_ref, w2a_ref, w2b_ref, o_ref):
    pv = p_ref[...]
    g0 = jnp.maximum(pv[0, 0] + pv[0, 1] + ba_ref[...], 0.0)
    g1 = jnp.maximum(pv[1, 0] + pv[1, 1] + bb_ref[...], 0.0)
    w2a = w2a_ref[...]
    w2b = w2b_ref[...]
    ev = (jnp.dot(g0[:, :64], w2a, preferred_element_type=jnp.float32)
          + jnp.dot(g1[:, :64], w2b, preferred_element_type=jnp.float32))
    ov = (jnp.dot(g0[:, 64:], w2a, preferred_element_type=jnp.float32)
          + jnp.dot(g1[:, 64:], w2b, preferred_element_type=jnp.float32))
    o_ref[...] = jnp.concatenate([ev, ov], axis=1)


def _tc_mid(p, ba, bb, w2a, w2b):
    return pl.pallas_call(
        _mid_body,
        out_shape=jax.ShapeDtypeStruct((NP // 2, 2 * F2), jnp.float32),
    )(p, ba, bb, w2a, w2b)


def _out_body(p_ref, b2_ref, o_ref):
    pv = p_ref[...]
    z2 = pv[0] + pv[1]
    b2v = b2_ref[...]
    outs = []
    for h in range(2):
        z = z2[:, h * F2:(h + 1) * F2] + b2v
        m = jnp.max(z, axis=1, keepdims=True)
        zs = z - m
        outs.append(zs - jnp.log(jnp.sum(jnp.exp(zs), axis=1, keepdims=True)))
    o_ref[...] = jnp.concatenate(outs, axis=1)


def _tc_out(p, b2):
    return pl.pallas_call(
        _out_body,
        out_shape=jax.ShapeDtypeStruct((NP // 2, 2 * F2), jnp.float32),
    )(p, b2)


@jax.jit
def kernel(x, edge_index, edge_weight, W1, b1, W2, b2):
    support = _tc_mm(x, W1)
    tab1 = support.reshape(2 * N, FW)
    p1 = _spmm1(edge_index, edge_weight, tab1)
    ba = jnp.tile(b1[:F2], 2).reshape(1, F1)
    bb = jnp.tile(b1[F2:], 2).reshape(1, F1)
    s2p = _tc_mid(p1.reshape(2, NC, NP // 2, 2 * FW),
                  ba, bb, W2[:F2], W2[F2:])
    tab2 = s2p.reshape(NP, FW)
    p2 = _spmm2(edge_index, edge_weight, tab2)
    o = _tc_out(p2.reshape(NC, NP // 2, 2 * FW), b2.reshape(1, F2))
    return o.reshape(NP, F2)[:N]
